# initial kernel scaffold (unmeasured)
import jax
import jax.numpy as jnp
from jax import lax
from jax.experimental import pallas as pl
from jax.experimental.pallas import tpu as pltpu


def kernel(
    x,
):
    def body(*refs):
        pass

    out_shape = jax.ShapeDtypeStruct(..., jnp.float32)
    return pl.pallas_call(body, out_shape=out_shape)(...)


# baseline (device time: 118007 ns/iter reference)
import jax
import jax.numpy as jnp
from jax import lax
from jax.experimental import pallas as pl
from jax.experimental.pallas import tpu as pltpu

N_DEV = 32


def kernel(x):
    _, m, n = x.shape
    ch = m // N_DEV

    def body(x_ref, out_ref, r_ref, rs_send, rs_recv, ag_send, ag_recv):
        my = lax.axis_index("i")

        bsem = pltpu.get_barrier_semaphore()
        for d in range(1, N_DEV):
            pl.semaphore_signal(
                bsem,
                inc=1,
                device_id=((my + d) % N_DEV,),
                device_id_type=pl.DeviceIdType.MESH,
            )
        pl.semaphore_wait(bsem, N_DEV - 1)

        a_rdmas = []
        for d in range(1, N_DEV):
            tgt = (my + d) % N_DEV
            rdma = pltpu.make_async_remote_copy(
                src_ref=x_ref.at[0, pl.ds(tgt * ch, ch), :],
                dst_ref=r_ref.at[d],
                send_sem=rs_send.at[d],
                recv_sem=rs_recv.at[d],
                device_id=(tgt,),
                device_id_type=pl.DeviceIdType.MESH,
            )
            rdma.start()
            a_rdmas.append(rdma)

        acc = x_ref[0, pl.ds(my * ch, ch), :]
        for d in range(1, N_DEV):
            a_rdmas[d - 1].wait_recv()
            acc = acc + r_ref[d]
        out_ref[pl.ds(my * ch, ch), :] = acc

        b_rdmas = []
        for d in range(1, N_DEV):
            tgt = (my + d) % N_DEV
            rdma = pltpu.make_async_remote_copy(
                src_ref=out_ref.at[pl.ds(my * ch, ch), :],
                dst_ref=out_ref.at[pl.ds(my * ch, ch), :],
                send_sem=ag_send.at[d],
                recv_sem=ag_recv.at[d],
                device_id=(tgt,),
                device_id_type=pl.DeviceIdType.MESH,
            )
            rdma.start()
            b_rdmas.append(rdma)

        for d in range(1, N_DEV):
            src = (my - d) % N_DEV
            recv = pltpu.make_async_remote_copy(
                src_ref=out_ref.at[pl.ds(src * ch, ch), :],
                dst_ref=out_ref.at[pl.ds(src * ch, ch), :],
                send_sem=ag_send.at[d],
                recv_sem=ag_recv.at[d],
                device_id=(src,),
                device_id_type=pl.DeviceIdType.MESH,
            )
            recv.wait_recv()

        for d in range(1, N_DEV):
            a_rdmas[d - 1].wait_send()
            b_rdmas[d - 1].wait_send()

    return pl.pallas_call(
        body,
        out_shape=jax.ShapeDtypeStruct((m, n), x.dtype),
        in_specs=[pl.BlockSpec(memory_space=pltpu.VMEM)],
        out_specs=pl.BlockSpec(memory_space=pltpu.VMEM),
        scratch_shapes=[
            pltpu.VMEM((N_DEV, ch, n), x.dtype),
            pltpu.SemaphoreType.DMA((N_DEV,)),
            pltpu.SemaphoreType.DMA((N_DEV,)),
            pltpu.SemaphoreType.DMA((N_DEV,)),
            pltpu.SemaphoreType.DMA((N_DEV,)),
        ],
        compiler_params=pltpu.CompilerParams(collective_id=0),
    )(x)


# device time: 110946 ns/iter; 1.0636x vs baseline; 1.0636x over previous
import jax
import jax.numpy as jnp
from jax import lax
from jax.experimental import pallas as pl
from jax.experimental.pallas import tpu as pltpu

N_DEV = 32
N_GRID = 16


def kernel(x):
    _, m, n = x.shape
    half = m // 2
    ch = m // N_DEV

    def body(
        x_ref,
        out_ref,
        p1_ref,
        h_ref,
        r_ref,
        x_sems,
        grs_send,
        grs_recv,
        gag_send,
        gag_recv,
    ):
        my = lax.axis_index("i")
        par = jnp.bitwise_and(my, 1)
        partner = jnp.bitwise_xor(my, 1)
        g = lax.shift_right_logical(my, 1)
        own_off = par * half + g * ch

        bsem = pltpu.get_barrier_semaphore()
        pl.semaphore_signal(
            bsem, inc=1, device_id=(partner,), device_id_type=pl.DeviceIdType.MESH
        )
        for k in range(1, N_GRID):
            peer = lax.rem(g + k, N_GRID) * 2 + par
            pl.semaphore_signal(
                bsem, inc=1, device_id=(peer,), device_id_type=pl.DeviceIdType.MESH
            )
        pl.semaphore_wait(bsem, N_GRID)

        x_rs = pltpu.make_async_remote_copy(
            src_ref=x_ref.at[0, pl.ds((1 - par) * half, half), :],
            dst_ref=p1_ref,
            send_sem=x_sems.at[0],
            recv_sem=x_sems.at[1],
            device_id=(partner,),
            device_id_type=pl.DeviceIdType.MESH,
        )
        x_rs.start()
        x_rs.wait_recv()
        h_ref[:, :] = x_ref[0, pl.ds(par * half, half), :] + p1_ref[:, :]

        a_rdmas = []
        for k in range(1, N_GRID):
            gp = lax.rem(g + k, N_GRID)
            peer = gp * 2 + par
            rdma = pltpu.make_async_remote_copy(
                src_ref=h_ref.at[pl.ds(gp * ch, ch), :],
                dst_ref=r_ref.at[k],
                send_sem=grs_send.at[k],
                recv_sem=grs_recv.at[k],
                device_id=(peer,),
                device_id_type=pl.DeviceIdType.MESH,
            )
            rdma.start()
            a_rdmas.append(rdma)

        acc = h_ref[pl.ds(g * ch, ch), :]
        for k in range(1, N_GRID):
            a_rdmas[k - 1].wait_recv()
            acc = acc + r_ref[k]
        out_ref[pl.ds(own_off, ch), :] = acc

        b_rdmas = []
        for k in range(1, N_GRID):
            peer = lax.rem(g + k, N_GRID) * 2 + par
            rdma = pltpu.make_async_remote_copy(
                src_ref=out_ref.at[pl.ds(own_off, ch), :],
                dst_ref=out_ref.at[pl.ds(own_off, ch), :],
                send_sem=gag_send.at[k],
                recv_sem=gag_recv.at[k],
                device_id=(peer,),
                device_id_type=pl.DeviceIdType.MESH,
            )
            rdma.start()
            b_rdmas.append(rdma)
        for k in range(1, N_GRID):
            src_off = par * half + lax.rem(g - k + N_GRID, N_GRID) * ch
            recv = pltpu.make_async_remote_copy(
                src_ref=out_ref.at[pl.ds(src_off, ch), :],
                dst_ref=out_ref.at[pl.ds(src_off, ch), :],
                send_sem=gag_send.at[k],
                recv_sem=gag_recv.at[k],
                device_id=(partner,),
                device_id_type=pl.DeviceIdType.MESH,
            )
            recv.wait_recv()

        x_ag = pltpu.make_async_remote_copy(
            src_ref=out_ref.at[pl.ds(par * half, half), :],
            dst_ref=out_ref.at[pl.ds(par * half, half), :],
            send_sem=x_sems.at[2],
            recv_sem=x_sems.at[3],
            device_id=(partner,),
            device_id_type=pl.DeviceIdType.MESH,
        )
        x_ag.start()
        x_ag_recv = pltpu.make_async_remote_copy(
            src_ref=out_ref.at[pl.ds((1 - par) * half, half), :],
            dst_ref=out_ref.at[pl.ds((1 - par) * half, half), :],
            send_sem=x_sems.at[2],
            recv_sem=x_sems.at[3],
            device_id=(partner,),
            device_id_type=pl.DeviceIdType.MESH,
        )
        x_ag_recv.wait_recv()

        x_rs.wait_send()
        x_ag.wait_send()
        for k in range(1, N_GRID):
            a_rdmas[k - 1].wait_send()
            b_rdmas[k - 1].wait_send()

    return pl.pallas_call(
        body,
        out_shape=jax.ShapeDtypeStruct((m, n), x.dtype),
        in_specs=[pl.BlockSpec(memory_space=pltpu.VMEM)],
        out_specs=pl.BlockSpec(memory_space=pltpu.VMEM),
        scratch_shapes=[
            pltpu.VMEM((half, n), x.dtype),
            pltpu.VMEM((half, n), x.dtype),
            pltpu.VMEM((N_GRID, ch, n), x.dtype),
            pltpu.SemaphoreType.DMA((4,)),
            pltpu.SemaphoreType.DMA((N_GRID,)),
            pltpu.SemaphoreType.DMA((N_GRID,)),
            pltpu.SemaphoreType.DMA((N_GRID,)),
            pltpu.SemaphoreType.DMA((N_GRID,)),
        ],
        compiler_params=pltpu.CompilerParams(collective_id=0),
    )(x)


# device time: 89174 ns/iter; 1.3233x vs baseline; 1.2442x over previous
import jax
import jax.numpy as jnp
from jax import lax
from jax.experimental import pallas as pl
from jax.experimental.pallas import tpu as pltpu

N_DEV = 32
N_GRID = 16


def kernel(x):
    _, m, n = x.shape
    half = m // 2
    ch = m // N_DEV

    def body(
        x_ref,
        out_ref,
        p1_ref,
        r_ref,
        xrs_send,
        xrs_recv,
        grs_send,
        grs_recv,
        gag_send,
        gag_recv,
        xag_send,
        xag_recv,
    ):
        my = lax.axis_index("i")
        par = lax.rem(my, 2)
        partner = my + 1 - 2 * par
        g = lax.div(my, 2)
        oth = 1 - par
        own_off = par * half + g * ch

        bsem = pltpu.get_barrier_semaphore()
        pl.semaphore_signal(
            bsem, inc=1, device_id=(partner,), device_id_type=pl.DeviceIdType.MESH
        )
        for k in range(1, N_GRID):
            peer = lax.rem(g + k, N_GRID) * 2 + par
            pl.semaphore_signal(
                bsem, inc=1, device_id=(peer,), device_id_type=pl.DeviceIdType.MESH
            )
        pl.semaphore_wait(bsem, N_GRID)

        xrs = []
        for j in range(1, N_GRID + 1):
            gp = lax.rem(g + j, N_GRID)
            rdma = pltpu.make_async_remote_copy(
                src_ref=x_ref.at[0, pl.ds(oth * half + gp * ch, ch), :],
                dst_ref=p1_ref.at[j],
                send_sem=xrs_send.at[j],
                recv_sem=xrs_recv.at[j],
                device_id=(partner,),
                device_id_type=pl.DeviceIdType.MESH,
            )
            rdma.start()
            xrs.append(rdma)

        grs = []
        for k in range(1, N_GRID):
            gp = lax.rem(g + k, N_GRID)
            peer = gp * 2 + par
            xrs[k - 1].wait_recv()
            p1_ref[k] = x_ref[0, pl.ds(par * half + gp * ch, ch), :] + p1_ref[k]
            rdma = pltpu.make_async_remote_copy(
                src_ref=p1_ref.at[k],
                dst_ref=r_ref.at[k],
                send_sem=grs_send.at[k],
                recv_sem=grs_recv.at[k],
                device_id=(peer,),
                device_id_type=pl.DeviceIdType.MESH,
            )
            rdma.start()
            grs.append(rdma)

        xrs[N_GRID - 1].wait_recv()
        acc = x_ref[0, pl.ds(par * half + g * ch, ch), :] + p1_ref[N_GRID]
        for k in range(1, N_GRID):
            grs[k - 1].wait_recv()
            acc = acc + r_ref[k]
        out_ref[pl.ds(own_off, ch), :] = acc

        xag0 = pltpu.make_async_remote_copy(
            src_ref=out_ref.at[pl.ds(own_off, ch), :],
            dst_ref=out_ref.at[pl.ds(own_off, ch), :],
            send_sem=xag_send.at[0],
            recv_sem=xag_recv.at[0],
            device_id=(partner,),
            device_id_type=pl.DeviceIdType.MESH,
        )
        xag0.start()
        gag = []
        for k in range(1, N_GRID):
            peer = lax.rem(g + k, N_GRID) * 2 + par
            rdma = pltpu.make_async_remote_copy(
                src_ref=out_ref.at[pl.ds(own_off, ch), :],
                dst_ref=out_ref.at[pl.ds(own_off, ch), :],
                send_sem=gag_send.at[k],
                recv_sem=gag_recv.at[k],
                device_id=(peer,),
                device_id_type=pl.DeviceIdType.MESH,
            )
            rdma.start()
            gag.append(rdma)

        xag = [xag0]
        for k in range(1, N_GRID):
            src_off = par * half + lax.rem(g - k + N_GRID, N_GRID) * ch
            recv = pltpu.make_async_remote_copy(
                src_ref=out_ref.at[pl.ds(src_off, ch), :],
                dst_ref=out_ref.at[pl.ds(src_off, ch), :],
                send_sem=gag_send.at[k],
                recv_sem=gag_recv.at[k],
                device_id=(partner,),
                device_id_type=pl.DeviceIdType.MESH,
            )
            recv.wait_recv()
            fwd = pltpu.make_async_remote_copy(
                src_ref=out_ref.at[pl.ds(src_off, ch), :],
                dst_ref=out_ref.at[pl.ds(src_off, ch), :],
                send_sem=xag_send.at[k],
                recv_sem=xag_recv.at[k],
                device_id=(partner,),
                device_id_type=pl.DeviceIdType.MESH,
            )
            fwd.start()
            xag.append(fwd)

        for j in range(N_GRID):
            off = oth * half + lax.rem(g - j + N_GRID, N_GRID) * ch
            recv = pltpu.make_async_remote_copy(
                src_ref=out_ref.at[pl.ds(off, ch), :],
                dst_ref=out_ref.at[pl.ds(off, ch), :],
                send_sem=xag_send.at[j],
                recv_sem=xag_recv.at[j],
                device_id=(partner,),
                device_id_type=pl.DeviceIdType.MESH,
            )
            recv.wait_recv()

        for r in xrs:
            r.wait_send()
        for r in grs:
            r.wait_send()
        for r in gag:
            r.wait_send()
        for r in xag:
            r.wait_send()

    return pl.pallas_call(
        body,
        out_shape=jax.ShapeDtypeStruct((m, n), x.dtype),
        in_specs=[pl.BlockSpec(memory_space=pltpu.VMEM)],
        out_specs=pl.BlockSpec(memory_space=pltpu.VMEM),
        scratch_shapes=[
            pltpu.VMEM((N_GRID + 1, ch, n), x.dtype),
            pltpu.VMEM((N_GRID, ch, n), x.dtype),
            pltpu.SemaphoreType.DMA((N_GRID + 1,)),
            pltpu.SemaphoreType.DMA((N_GRID + 1,)),
            pltpu.SemaphoreType.DMA((N_GRID,)),
            pltpu.SemaphoreType.DMA((N_GRID,)),
            pltpu.SemaphoreType.DMA((N_GRID,)),
            pltpu.SemaphoreType.DMA((N_GRID,)),
            pltpu.SemaphoreType.DMA((N_GRID,)),
            pltpu.SemaphoreType.DMA((N_GRID,)),
        ],
        compiler_params=pltpu.CompilerParams(collective_id=0),
    )(x)


# device time: 83631 ns/iter; 1.4110x vs baseline; 1.0663x over previous
import jax
import jax.numpy as jnp
from jax import lax
from jax.experimental import pallas as pl
from jax.experimental.pallas import tpu as pltpu

N_DEV = 32
N_GRID = 16
N_WAVE = 2


def kernel(x):
    _, m, n = x.shape
    half = m // 2
    ch = m // N_DEV
    hc = ch // N_WAVE

    def body(
        x_ref,
        out_ref,
        p1_ref,
        r_ref,
        xrs_send,
        xrs_recv,
        grs_send,
        grs_recv,
        gag_send,
        gag_recv,
        xag_send,
        xag_recv,
    ):
        my = lax.axis_index("i")
        par = lax.rem(my, 2)
        partner = my + 1 - 2 * par
        g = lax.div(my, 2)
        oth = 1 - par
        own_off = par * half + g * ch

        bsem = pltpu.get_barrier_semaphore()
        pl.semaphore_signal(
            bsem, inc=1, device_id=(partner,), device_id_type=pl.DeviceIdType.MESH
        )
        for k in range(1, N_GRID):
            peer = lax.rem(g + k, N_GRID) * 2 + par
            pl.semaphore_signal(
                bsem, inc=1, device_id=(peer,), device_id_type=pl.DeviceIdType.MESH
            )
        pl.semaphore_wait(bsem, N_GRID)

        xrs = {}
        for w in range(N_WAVE):
            for j in range(1, N_GRID + 1):
                gp = lax.rem(g + j, N_GRID)
                rdma = pltpu.make_async_remote_copy(
                    src_ref=x_ref.at[
                        0, pl.ds(oth * half + gp * ch + w * hc, hc), :
                    ],
                    dst_ref=p1_ref.at[j, pl.ds(w * hc, hc), :],
                    send_sem=xrs_send.at[w, j],
                    recv_sem=xrs_recv.at[w, j],
                    device_id=(partner,),
                    device_id_type=pl.DeviceIdType.MESH,
                )
                rdma.start()
                xrs[(w, j)] = rdma

        grs = {}
        gag = {}
        xag = {}
        for w in range(N_WAVE):
            for k in range(1, N_GRID):
                gp = lax.rem(g + k, N_GRID)
                peer = gp * 2 + par
                xrs[(w, k)].wait_recv()
                p1_ref[k, pl.ds(w * hc, hc), :] = (
                    x_ref[0, pl.ds(par * half + gp * ch + w * hc, hc), :]
                    + p1_ref[k, pl.ds(w * hc, hc), :]
                )
                rdma = pltpu.make_async_remote_copy(
                    src_ref=p1_ref.at[k, pl.ds(w * hc, hc), :],
                    dst_ref=r_ref.at[k, pl.ds(w * hc, hc), :],
                    send_sem=grs_send.at[w, k],
                    recv_sem=grs_recv.at[w, k],
                    device_id=(peer,),
                    device_id_type=pl.DeviceIdType.MESH,
                )
                rdma.start()
                grs[(w, k)] = rdma

            xrs[(w, N_GRID)].wait_recv()
            acc = (
                x_ref[0, pl.ds(par * half + g * ch + w * hc, hc), :]
                + p1_ref[N_GRID, pl.ds(w * hc, hc), :]
            )
            for k in range(1, N_GRID):
                grs[(w, k)].wait_recv()
                acc = acc + r_ref[k, pl.ds(w * hc, hc), :]
            out_ref[pl.ds(own_off + w * hc, hc), :] = acc

            fwd0 = pltpu.make_async_remote_copy(
                src_ref=out_ref.at[pl.ds(own_off + w * hc, hc), :],
                dst_ref=out_ref.at[pl.ds(own_off + w * hc, hc), :],
                send_sem=xag_send.at[w, 0],
                recv_sem=xag_recv.at[w, 0],
                device_id=(partner,),
                device_id_type=pl.DeviceIdType.MESH,
            )
            fwd0.start()
            xag[(w, 0)] = fwd0
            for k in range(1, N_GRID):
                peer = lax.rem(g + k, N_GRID) * 2 + par
                rdma = pltpu.make_async_remote_copy(
                    src_ref=out_ref.at[pl.ds(own_off + w * hc, hc), :],
                    dst_ref=out_ref.at[pl.ds(own_off + w * hc, hc), :],
                    send_sem=gag_send.at[w, k],
                    recv_sem=gag_recv.at[w, k],
                    device_id=(peer,),
                    device_id_type=pl.DeviceIdType.MESH,
                )
                rdma.start()
                gag[(w, k)] = rdma

        for w in range(N_WAVE):
            for k in range(1, N_GRID):
                src_off = (
                    par * half + lax.rem(g - k + N_GRID, N_GRID) * ch + w * hc
                )
                recv = pltpu.make_async_remote_copy(
                    src_ref=out_ref.at[pl.ds(src_off, hc), :],
                    dst_ref=out_ref.at[pl.ds(src_off, hc), :],
                    send_sem=gag_send.at[w, k],
                    recv_sem=gag_recv.at[w, k],
                    device_id=(partner,),
                    device_id_type=pl.DeviceIdType.MESH,
                )
                recv.wait_recv()
                fwd = pltpu.make_async_remote_copy(
                    src_ref=out_ref.at[pl.ds(src_off, hc), :],
                    dst_ref=out_ref.at[pl.ds(src_off, hc), :],
                    send_sem=xag_send.at[w, k],
                    recv_sem=xag_recv.at[w, k],
                    device_id=(partner,),
                    device_id_type=pl.DeviceIdType.MESH,
                )
                fwd.start()
                xag[(w, k)] = fwd

        for w in range(N_WAVE):
            for j in range(N_GRID):
                off = (
                    oth * half
                    + lax.rem(g - j + N_GRID, N_GRID) * ch
                    + w * hc
                )
                recv = pltpu.make_async_remote_copy(
                    src_ref=out_ref.at[pl.ds(off, hc), :],
                    dst_ref=out_ref.at[pl.ds(off, hc), :],
                    send_sem=xag_send.at[w, j],
                    recv_sem=xag_recv.at[w, j],
                    device_id=(partner,),
                    device_id_type=pl.DeviceIdType.MESH,
                )
                recv.wait_recv()

        for r in xrs.values():
            r.wait_send()
        for r in grs.values():
            r.wait_send()
        for r in gag.values():
            r.wait_send()
        for r in xag.values():
            r.wait_send()

    return pl.pallas_call(
        body,
        out_shape=jax.ShapeDtypeStruct((m, n), x.dtype),
        in_specs=[pl.BlockSpec(memory_space=pltpu.VMEM)],
        out_specs=pl.BlockSpec(memory_space=pltpu.VMEM),
        scratch_shapes=[
            pltpu.VMEM((N_GRID + 1, ch, n), x.dtype),
            pltpu.VMEM((N_GRID, ch, n), x.dtype),
            pltpu.SemaphoreType.DMA((N_WAVE, N_GRID + 1)),
            pltpu.SemaphoreType.DMA((N_WAVE, N_GRID + 1)),
            pltpu.SemaphoreType.DMA((N_WAVE, N_GRID)),
            pltpu.SemaphoreType.DMA((N_WAVE, N_GRID)),
            pltpu.SemaphoreType.DMA((N_WAVE, N_GRID)),
            pltpu.SemaphoreType.DMA((N_WAVE, N_GRID)),
            pltpu.SemaphoreType.DMA((N_WAVE, N_GRID)),
            pltpu.SemaphoreType.DMA((N_WAVE, N_GRID)),
        ],
        compiler_params=pltpu.CompilerParams(collective_id=0),
    )(x)
